# baseline (speedup 1.0000x reference)
import functools

import jax
import jax.numpy as jnp
from jax import lax
from jax.experimental import pallas as pl
from jax.experimental.pallas import tpu as pltpu
from jax.experimental.pallas import tpu_sc as plsc

N = 10000
E = 320000
D_IN = 128
D_H = 64
G = 128
HN = N // 2

NC = 2
NS = 16
NW = NC * NS
EPW = E // NW
K = 80
NCHUNK = EPW // K
RPT = 624
RPT_LAST = N - RPT * (NS - 1)
RB = 2000
PB = RB // 2
GRID = N // RB
NBUF = 5
GROUPS = NCHUNK // NBUF


def _mesh():
    return plsc.VectorSubcoreMesh(
        core_axis_name="c", subcore_axis_name="s", num_cores=NC, num_subcores=NS
    )


_SC_PARAMS = pltpu.CompilerParams(use_tc_tiling_on_sc=False)


@functools.partial(
    pl.kernel,
    out_type=(
        jax.ShapeDtypeStruct((GRID, NC, PB), jnp.float32),
        jax.ShapeDtypeStruct((GRID, NC, PB), jnp.float32),
    ),
    mesh=_mesh(),
    scratch_types=[
        pltpu.VMEM((NCHUNK, K), jnp.int32),
        pltpu.VMEM((K,), jnp.float32),
        pltpu.VMEM_SHARED((N,), jnp.float32),
        pltpu.SemaphoreType.DMA,
    ],
    compiler_params=_SC_PARAMS,
)
def _sc_degree(dst_hbm, zeros_hbm, oute_hbm, outo_hbm, di_big, ones_v, sdeg,
               sem):
    c = lax.axis_index("c")
    s = lax.axis_index("s")
    wid = c * NS + s
    for j in range(K // 16):
        ones_v[pl.ds(j * 16, 16)] = jnp.ones((16,), jnp.float32)
    pltpu.sync_copy(dst_hbm.at[wid], di_big)

    @pl.when(s == 0)
    def _():
        pltpu.sync_copy(zeros_hbm, sdeg)

    plsc.subcore_barrier()

    def body(g, carry):
        descs = [
            pltpu.async_copy(ones_v, sdeg.at[di_big.at[g * NBUF + b]], sem,
                             add=True)
            for b in range(NBUF)
        ]
        for d in descs:
            d.wait()
        return carry

    lax.fori_loop(0, GROUPS, body, 0)
    plsc.subcore_barrier()

    @pl.when(s == 0)
    def _():
        for g in range(GRID):
            pltpu.sync_copy(sdeg.at[pl.ds(g * PB, PB)], oute_hbm.at[g, c])
            pltpu.sync_copy(sdeg.at[pl.ds(HN + g * PB, PB)], outo_hbm.at[g, c])


@functools.partial(
    pl.kernel,
    out_type=jax.ShapeDtypeStruct((NC, N, D_H), jnp.float32),
    mesh=_mesh(),
    scratch_types=[
        pltpu.VMEM((NCHUNK, K), jnp.int32),
        pltpu.VMEM((NCHUNK, K), jnp.int32),
        pltpu.VMEM((2 * NBUF, K, D_H), jnp.float32),
        pltpu.VMEM_SHARED((N, D_H), jnp.float32),
        [pltpu.SemaphoreType.DMA] * (2 * NBUF),
        [pltpu.SemaphoreType.DMA] * (2 * NBUF),
    ],
    compiler_params=_SC_PARAMS,
)
def _sc_aggregate(y_hbm, src_hbm, dst_hbm, zrows_hbm, out_hbm,
                  si_big, di_big, rows_v, sacc, gsems, ssems):
    c = lax.axis_index("c")
    s = lax.axis_index("s")
    wid = c * NS + s
    pltpu.sync_copy(src_hbm.at[wid], si_big)
    pltpu.sync_copy(dst_hbm.at[wid], di_big)

    @pl.when(s < NS - 1)
    def _():
        r0 = s * RPT
        pltpu.sync_copy(zrows_hbm.at[pl.ds(r0, RPT)], sacc.at[pl.ds(r0, RPT)])

    @pl.when(s == NS - 1)
    def _():
        r0 = (NS - 1) * RPT
        pltpu.sync_copy(zrows_hbm.at[pl.ds(r0, RPT_LAST)],
                        sacc.at[pl.ds(r0, RPT_LAST)])

    plsc.subcore_barrier()

    def _gather(j, b):
        return pltpu.async_copy(y_hbm.at[si_big.at[j]], rows_v.at[b],
                                gsems[b])

    def _gather_wait(j, b):
        pltpu.make_async_copy(y_hbm.at[si_big.at[j]], rows_v.at[b],
                              gsems[b]).wait()

    def _scatter(j, b):
        return pltpu.async_copy(rows_v.at[b], sacc.at[di_big.at[j]],
                                ssems[b], add=True)

    for b in range(NBUF):
        _gather(b, b)
    for b in range(NBUF):
        _gather(NBUF + b, NBUF + b)

    def _process(g, half_base):
        sds = []
        for b in range(NBUF):
            _gather_wait(g * NBUF + b, half_base + b)
            sds.append(_scatter(g * NBUF + b, half_base + b))
        for b in range(NBUF):
            sds[b].wait()

            @pl.when(g + 2 < GROUPS)
            def _():
                _gather((g + 2) * NBUF + b, half_base + b)

    def body(i, carry):
        _process(2 * i, 0)
        _process(2 * i + 1, NBUF)
        return carry

    lax.fori_loop(0, GROUPS // 2, body, 0)
    _process(GROUPS - 1, 0)
    plsc.subcore_barrier()

    @pl.when(s < NS - 1)
    def _():
        r0 = s * RPT
        pltpu.sync_copy(sacc.at[pl.ds(r0, RPT)], out_hbm.at[c, pl.ds(r0, RPT)])

    @pl.when(s == NS - 1)
    def _():
        r0 = (NS - 1) * RPT
        pltpu.sync_copy(sacc.at[pl.ds(r0, RPT_LAST)],
                        out_hbm.at[c, pl.ds(r0, RPT_LAST)])


def _dscale(dge_blk, dgo_blk):
    ev = 1.0 + dge_blk[0, 0:1, :] + dge_blk[0, 1:2, :]
    od = 1.0 + dgo_blk[0, 0:1, :] + dgo_blk[0, 1:2, :]
    de = jnp.transpose(lax.rsqrt(ev), (1, 0))
    do = jnp.transpose(lax.rsqrt(od), (1, 0))
    return jnp.concatenate(
        [jnp.broadcast_to(de, (PB, D_H)), jnp.broadcast_to(do, (PB, D_H))],
        axis=1,
    )


def _tc1_body(xp_ref, wd_ref, dge_ref, dgo_ref, y_ref):
    ds_blk = _dscale(dge_ref[...], dgo_ref[...])
    y_ref[...] = jnp.dot(xp_ref[...], wd_ref[...],
                         preferred_element_type=jnp.float32) * ds_blk


_tc1 = pl.pallas_call(
    _tc1_body,
    grid=(GRID,),
    in_specs=[
        pl.BlockSpec((PB, 2 * D_IN), lambda i: (i, 0)),
        pl.BlockSpec((2 * D_IN, 2 * D_H), lambda i: (0, 0)),
        pl.BlockSpec((1, NC, PB), lambda i: (i, 0, 0)),
        pl.BlockSpec((1, NC, PB), lambda i: (i, 0, 0)),
    ],
    out_specs=pl.BlockSpec((PB, 2 * D_H), lambda i: (i, 0)),
    out_shape=jax.ShapeDtypeStruct((HN, 2 * D_H), jnp.float32),
)


def _tc2_body(pq_ref, y1_ref, dge_ref, dgo_ref, b_ref, wd_ref, out_ref):
    ds_blk = _dscale(dge_ref[...], dgo_ref[...])
    s = pq_ref[0] + pq_ref[1] + y1_ref[...]
    h = jnp.maximum(ds_blk * s + b_ref[...], 0.0)
    out_ref[...] = jnp.dot(h, wd_ref[...],
                           preferred_element_type=jnp.float32) * ds_blk


_tc2 = pl.pallas_call(
    _tc2_body,
    grid=(GRID,),
    in_specs=[
        pl.BlockSpec((NC, PB, 2 * D_H), lambda i: (0, i, 0)),
        pl.BlockSpec((PB, 2 * D_H), lambda i: (i, 0)),
        pl.BlockSpec((1, NC, PB), lambda i: (i, 0, 0)),
        pl.BlockSpec((1, NC, PB), lambda i: (i, 0, 0)),
        pl.BlockSpec((1, 2 * D_H), lambda i: (0, 0)),
        pl.BlockSpec((2 * D_H, 2 * D_H), lambda i: (0, 0)),
    ],
    out_specs=pl.BlockSpec((PB, 2 * D_H), lambda i: (i, 0)),
    out_shape=jax.ShapeDtypeStruct((HN, 2 * D_H), jnp.float32),
)


def _tc3_body(qq_ref, y2_ref, dge_ref, dgo_ref, b_ref, be_ref, bo_ref,
              out_ref, acc_ref):
    i = pl.program_id(0)
    ds_blk = _dscale(dge_ref[...], dgo_ref[...])
    o2 = ds_blk * (qq_ref[0] + qq_ref[1] + y2_ref[...]) + b_ref[...]
    ones = jnp.ones((PB, D_H), jnp.float32)
    gid = lax.broadcasted_iota(jnp.int32, (PB, G), 1)
    bce = jnp.transpose(be_ref[0], (1, 0))
    bco = jnp.transpose(bo_ref[0], (1, 0))
    oh_e = (bce == gid).astype(jnp.float32)
    oh_o = (bco == gid).astype(jnp.float32)
    ext_e = jnp.concatenate([o2[:, :D_H], ones], axis=1)
    ext_o = jnp.concatenate([o2[:, D_H:], ones], axis=1)
    dn = (((0,), (0,)), ((), ()))
    part = (lax.dot_general(oh_e, ext_e, dn, preferred_element_type=jnp.float32)
            + lax.dot_general(oh_o, ext_o, dn,
                              preferred_element_type=jnp.float32))

    @pl.when(i == 0)
    def _():
        acc_ref[...] = jnp.zeros_like(acc_ref)

    acc_ref[...] += part

    @pl.when(i == pl.num_programs(0) - 1)
    def _():
        a = acc_ref[...]
        out_ref[...] = a[:, :D_H] / jnp.maximum(a[:, D_H:D_H + 1], 1.0)


_tc3 = pl.pallas_call(
    _tc3_body,
    grid=(GRID,),
    in_specs=[
        pl.BlockSpec((NC, PB, 2 * D_H), lambda i: (0, i, 0)),
        pl.BlockSpec((PB, 2 * D_H), lambda i: (i, 0)),
        pl.BlockSpec((1, NC, PB), lambda i: (i, 0, 0)),
        pl.BlockSpec((1, NC, PB), lambda i: (i, 0, 0)),
        pl.BlockSpec((1, 2 * D_H), lambda i: (0, 0)),
        pl.BlockSpec((1, 1, PB), lambda i: (i, 0, 0)),
        pl.BlockSpec((1, 1, PB), lambda i: (i, 0, 0)),
    ],
    out_specs=pl.BlockSpec((G, D_H), lambda i: (0, 0)),
    out_shape=jax.ShapeDtypeStruct((G, D_H), jnp.float32),
    scratch_shapes=[pltpu.VMEM((G, G), jnp.float32)],
)


def _blockdiag(W):
    z = jnp.zeros_like(W)
    return jnp.concatenate(
        [jnp.concatenate([W, z], axis=1), jnp.concatenate([z, W], axis=1)],
        axis=0,
    )


def kernel(x, edge_index, batch, W1, b1, W2, b2):
    src = edge_index[0].reshape(NW, NCHUNK, K)
    dst = edge_index[1].reshape(NW, NCHUNK, K)
    dstp = (dst >> 1) + (dst & 1) * HN
    zdeg = jnp.zeros((N,), jnp.float32)
    zrow = jnp.zeros((N, D_H), jnp.float32)
    xpair = x.reshape(HN, 2 * D_IN)
    W1d = _blockdiag(W1)
    W2d = _blockdiag(W2)
    b1p = jnp.concatenate([b1, b1]).reshape(1, 2 * D_H)
    b2p = jnp.concatenate([b2, b2]).reshape(1, 2 * D_H)
    bev = batch[0::2].reshape(GRID, 1, PB)
    bod = batch[1::2].reshape(GRID, 1, PB)

    dge, dgo = _sc_degree(dstp, zdeg)
    y1p = _tc1(xpair, W1d, dge, dgo)
    p = _sc_aggregate(y1p.reshape(N, D_H), src, dst, zrow)
    y2p = _tc2(p.reshape(NC, HN, 2 * D_H), y1p, dge, dgo, b1p, W2d)
    q = _sc_aggregate(y2p.reshape(N, D_H), src, dst, zrow)
    out = _tc3(q.reshape(NC, HN, 2 * D_H), y2p, dge, dgo, b2p, bev, bod)
    return out


# speedup vs baseline: 1.0141x; 1.0141x over previous
import functools

import jax
import jax.numpy as jnp
from jax import lax
from jax.experimental import pallas as pl
from jax.experimental.pallas import tpu as pltpu
from jax.experimental.pallas import tpu_sc as plsc

N = 10000
E = 320000
D_IN = 128
D_H = 64
G = 128
HN = N // 2

NC = 2
NS = 16
NW = NC * NS
EPW = E // NW
K = 80
NCHUNK = EPW // K
RPT = 624
RPT_LAST = N - RPT * (NS - 1)
RB = 2000
PB = RB // 2
GRID = N // RB
NBUF = 5
GROUPS = NCHUNK // NBUF


def _mesh():
    return plsc.VectorSubcoreMesh(
        core_axis_name="c", subcore_axis_name="s", num_cores=NC, num_subcores=NS
    )


_SC_PARAMS = pltpu.CompilerParams(use_tc_tiling_on_sc=False)


@functools.partial(
    pl.kernel,
    out_type=(
        jax.ShapeDtypeStruct((GRID, NC, PB), jnp.float32),
        jax.ShapeDtypeStruct((GRID, NC, PB), jnp.float32),
    ),
    mesh=_mesh(),
    scratch_types=[
        pltpu.VMEM((NCHUNK, K), jnp.int32),
        pltpu.VMEM((K,), jnp.float32),
        pltpu.VMEM_SHARED((N,), jnp.float32),
        pltpu.SemaphoreType.DMA,
    ],
    compiler_params=_SC_PARAMS,
)
def _sc_degree(dst_hbm, zeros_hbm, oute_hbm, outo_hbm, di_big, ones_v, sdeg,
               sem):
    c = lax.axis_index("c")
    s = lax.axis_index("s")
    wid = c * NS + s
    for j in range(K // 16):
        ones_v[pl.ds(j * 16, 16)] = jnp.ones((16,), jnp.float32)
    pltpu.sync_copy(dst_hbm.at[wid], di_big)

    @plsc.parallel_loop(0, NCHUNK, 1, unroll=5)
    def _(j):
        for l in range(K // 16):
            v = di_big[j, pl.ds(l * 16, 16)]
            di_big[j, pl.ds(l * 16, 16)] = (v >> 1) + (v & 1) * HN

    @pl.when(s == 0)
    def _():
        pltpu.sync_copy(zeros_hbm, sdeg)

    plsc.subcore_barrier()

    def body(g, carry):
        descs = [
            pltpu.async_copy(ones_v, sdeg.at[di_big.at[g * NBUF + b]], sem,
                             add=True)
            for b in range(NBUF)
        ]
        for d in descs:
            d.wait()
        return carry

    lax.fori_loop(0, GROUPS, body, 0)
    plsc.subcore_barrier()

    @pl.when(s == 0)
    def _():
        for g in range(GRID):
            pltpu.sync_copy(sdeg.at[pl.ds(g * PB, PB)], oute_hbm.at[g, c])
            pltpu.sync_copy(sdeg.at[pl.ds(HN + g * PB, PB)], outo_hbm.at[g, c])


@functools.partial(
    pl.kernel,
    out_type=jax.ShapeDtypeStruct((NC, N, D_H), jnp.float32),
    mesh=_mesh(),
    scratch_types=[
        pltpu.VMEM((NCHUNK, K), jnp.int32),
        pltpu.VMEM((NCHUNK, K), jnp.int32),
        pltpu.VMEM((2 * NBUF, K, D_H), jnp.float32),
        pltpu.VMEM_SHARED((N, D_H), jnp.float32),
        [pltpu.SemaphoreType.DMA] * (2 * NBUF),
        [pltpu.SemaphoreType.DMA] * (2 * NBUF),
    ],
    compiler_params=_SC_PARAMS,
)
def _sc_aggregate(y_hbm, src_hbm, dst_hbm, zrows_hbm, out_hbm,
                  si_big, di_big, rows_v, sacc, gsems, ssems):
    c = lax.axis_index("c")
    s = lax.axis_index("s")
    wid = c * NS + s
    pltpu.sync_copy(src_hbm.at[wid], si_big)
    pltpu.sync_copy(dst_hbm.at[wid], di_big)

    @pl.when(s < NS - 1)
    def _():
        r0 = s * RPT
        pltpu.sync_copy(zrows_hbm.at[pl.ds(r0, RPT)], sacc.at[pl.ds(r0, RPT)])

    @pl.when(s == NS - 1)
    def _():
        r0 = (NS - 1) * RPT
        pltpu.sync_copy(zrows_hbm.at[pl.ds(r0, RPT_LAST)],
                        sacc.at[pl.ds(r0, RPT_LAST)])

    plsc.subcore_barrier()

    def _gather(j, b):
        return pltpu.async_copy(y_hbm.at[si_big.at[j]], rows_v.at[b],
                                gsems[b])

    def _gather_wait(j, b):
        pltpu.make_async_copy(y_hbm.at[si_big.at[j]], rows_v.at[b],
                              gsems[b]).wait()

    def _scatter(j, b):
        return pltpu.async_copy(rows_v.at[b], sacc.at[di_big.at[j]],
                                ssems[b], add=True)

    for b in range(NBUF):
        _gather(b, b)
    for b in range(NBUF):
        _gather(NBUF + b, NBUF + b)

    def _process(g, half_base):
        sds = []
        for b in range(NBUF):
            _gather_wait(g * NBUF + b, half_base + b)
            sds.append(_scatter(g * NBUF + b, half_base + b))
        for b in range(NBUF):
            sds[b].wait()

            @pl.when(g + 2 < GROUPS)
            def _():
                _gather((g + 2) * NBUF + b, half_base + b)

    def body(i, carry):
        _process(2 * i, 0)
        _process(2 * i + 1, NBUF)
        return carry

    lax.fori_loop(0, GROUPS // 2, body, 0)
    _process(GROUPS - 1, 0)
    plsc.subcore_barrier()

    @pl.when(s < NS - 1)
    def _():
        r0 = s * RPT
        pltpu.sync_copy(sacc.at[pl.ds(r0, RPT)], out_hbm.at[c, pl.ds(r0, RPT)])

    @pl.when(s == NS - 1)
    def _():
        r0 = (NS - 1) * RPT
        pltpu.sync_copy(sacc.at[pl.ds(r0, RPT_LAST)],
                        out_hbm.at[c, pl.ds(r0, RPT_LAST)])


def _dscale(dge_blk, dgo_blk):
    ev = 1.0 + dge_blk[0, 0:1, :] + dge_blk[0, 1:2, :]
    od = 1.0 + dgo_blk[0, 0:1, :] + dgo_blk[0, 1:2, :]
    de = jnp.transpose(lax.rsqrt(ev), (1, 0))
    do = jnp.transpose(lax.rsqrt(od), (1, 0))
    return jnp.concatenate(
        [jnp.broadcast_to(de, (PB, D_H)), jnp.broadcast_to(do, (PB, D_H))],
        axis=1,
    )


def _tc1_body(xp_ref, wd_ref, dge_ref, dgo_ref, y_ref):
    ds_blk = _dscale(dge_ref[...], dgo_ref[...])
    y_ref[...] = jnp.dot(xp_ref[...], wd_ref[...],
                         preferred_element_type=jnp.float32) * ds_blk


_tc1 = pl.pallas_call(
    _tc1_body,
    grid=(GRID,),
    in_specs=[
        pl.BlockSpec((PB, 2 * D_IN), lambda i: (i, 0)),
        pl.BlockSpec((2 * D_IN, 2 * D_H), lambda i: (0, 0)),
        pl.BlockSpec((1, NC, PB), lambda i: (i, 0, 0)),
        pl.BlockSpec((1, NC, PB), lambda i: (i, 0, 0)),
    ],
    out_specs=pl.BlockSpec((PB, 2 * D_H), lambda i: (i, 0)),
    out_shape=jax.ShapeDtypeStruct((HN, 2 * D_H), jnp.float32),
)


def _tc2_body(pq_ref, y1_ref, dge_ref, dgo_ref, b_ref, wd_ref, out_ref):
    ds_blk = _dscale(dge_ref[...], dgo_ref[...])
    s = pq_ref[0] + pq_ref[1] + y1_ref[...]
    h = jnp.maximum(ds_blk * s + b_ref[...], 0.0)
    out_ref[...] = jnp.dot(h, wd_ref[...],
                           preferred_element_type=jnp.float32) * ds_blk


_tc2 = pl.pallas_call(
    _tc2_body,
    grid=(GRID,),
    in_specs=[
        pl.BlockSpec((NC, PB, 2 * D_H), lambda i: (0, i, 0)),
        pl.BlockSpec((PB, 2 * D_H), lambda i: (i, 0)),
        pl.BlockSpec((1, NC, PB), lambda i: (i, 0, 0)),
        pl.BlockSpec((1, NC, PB), lambda i: (i, 0, 0)),
        pl.BlockSpec((1, 2 * D_H), lambda i: (0, 0)),
        pl.BlockSpec((2 * D_H, 2 * D_H), lambda i: (0, 0)),
    ],
    out_specs=pl.BlockSpec((PB, 2 * D_H), lambda i: (i, 0)),
    out_shape=jax.ShapeDtypeStruct((HN, 2 * D_H), jnp.float32),
)


def _tc3_body(qq_ref, y2_ref, dge_ref, dgo_ref, b_ref, be_ref, bo_ref,
              out_ref, acc_ref):
    i = pl.program_id(0)
    ds_blk = _dscale(dge_ref[...], dgo_ref[...])
    o2 = ds_blk * (qq_ref[0] + qq_ref[1] + y2_ref[...]) + b_ref[...]
    ones = jnp.ones((PB, D_H), jnp.float32)
    gid = lax.broadcasted_iota(jnp.int32, (PB, G), 1)
    bce = jnp.transpose(be_ref[0], (1, 0))
    bco = jnp.transpose(bo_ref[0], (1, 0))
    oh_e = (bce == gid).astype(jnp.float32)
    oh_o = (bco == gid).astype(jnp.float32)
    ext_e = jnp.concatenate([o2[:, :D_H], ones], axis=1)
    ext_o = jnp.concatenate([o2[:, D_H:], ones], axis=1)
    dn = (((0,), (0,)), ((), ()))
    part = (lax.dot_general(oh_e, ext_e, dn, preferred_element_type=jnp.float32)
            + lax.dot_general(oh_o, ext_o, dn,
                              preferred_element_type=jnp.float32))

    @pl.when(i == 0)
    def _():
        acc_ref[...] = jnp.zeros_like(acc_ref)

    acc_ref[...] += part

    @pl.when(i == pl.num_programs(0) - 1)
    def _():
        a = acc_ref[...]
        out_ref[...] = a[:, :D_H] / jnp.maximum(a[:, D_H:D_H + 1], 1.0)


_tc3 = pl.pallas_call(
    _tc3_body,
    grid=(GRID,),
    in_specs=[
        pl.BlockSpec((NC, PB, 2 * D_H), lambda i: (0, i, 0)),
        pl.BlockSpec((PB, 2 * D_H), lambda i: (i, 0)),
        pl.BlockSpec((1, NC, PB), lambda i: (i, 0, 0)),
        pl.BlockSpec((1, NC, PB), lambda i: (i, 0, 0)),
        pl.BlockSpec((1, 2 * D_H), lambda i: (0, 0)),
        pl.BlockSpec((1, 1, PB), lambda i: (i, 0, 0)),
        pl.BlockSpec((1, 1, PB), lambda i: (i, 0, 0)),
    ],
    out_specs=pl.BlockSpec((G, D_H), lambda i: (0, 0)),
    out_shape=jax.ShapeDtypeStruct((G, D_H), jnp.float32),
    scratch_shapes=[pltpu.VMEM((G, G), jnp.float32)],
)


def _blockdiag(W):
    z = jnp.zeros_like(W)
    return jnp.concatenate(
        [jnp.concatenate([W, z], axis=1), jnp.concatenate([z, W], axis=1)],
        axis=0,
    )


def kernel(x, edge_index, batch, W1, b1, W2, b2):
    src = edge_index[0].reshape(NW, NCHUNK, K)
    dst = edge_index[1].reshape(NW, NCHUNK, K)
    zdeg = jnp.zeros((N,), jnp.float32)
    zrow = jnp.zeros((N, D_H), jnp.float32)
    xpair = x.reshape(HN, 2 * D_IN)
    W1d = _blockdiag(W1)
    W2d = _blockdiag(W2)
    b1p = jnp.concatenate([b1, b1]).reshape(1, 2 * D_H)
    b2p = jnp.concatenate([b2, b2]).reshape(1, 2 * D_H)
    bev = batch[0::2].reshape(GRID, 1, PB)
    bod = batch[1::2].reshape(GRID, 1, PB)

    dge, dgo = _sc_degree(dst, zdeg)
    y1p = _tc1(xpair, W1d, dge, dgo)
    p = _sc_aggregate(y1p.reshape(N, D_H), src, dst, zrow)
    y2p = _tc2(p.reshape(NC, HN, 2 * D_H), y1p, dge, dgo, b1p, W2d)
    q = _sc_aggregate(y2p.reshape(N, D_H), src, dst, zrow)
    out = _tc3(q.reshape(NC, HN, 2 * D_H), y2p, dge, dgo, b2p, bev, bod)
    return out
